# Initial kernel scaffold; baseline (speedup 1.0000x reference)
import jax
import jax.numpy as jnp
from jax.experimental import pallas as pl

_OUTPUT_SIZE = 7
_SPATIAL_SCALE = 0.25
_SAMPLING_RATIO = 2
_G = 32


def _weights(coords, limit, size):
    G = coords.shape[0]
    P = _OUTPUT_SIZE
    S = _SAMPLING_RATIO
    valid = (coords >= -1.0) & (coords <= float(limit))
    cc = jnp.clip(coords, 0.0, float(limit - 1))
    axis = jax.lax.broadcasted_iota(jnp.float32, (G, P * S, size), 2)
    w = jnp.maximum(1.0 - jnp.abs(axis - cc[:, :, None]), 0.0)
    w = jnp.where(valid[:, :, None], w, 0.0)
    w = w.reshape(G, P, S, size).sum(axis=2) * (1.0 / S)
    return w


def _roi_kernel(rois_ref, feat_ref, out_ref):
    G = _G
    P = _OUTPUT_SIZE
    N, H, W, C = 4, 56, 56, 256

    rois = rois_ref[...]
    b = rois[:, 0:1]
    x1 = rois[:, 1:2] * _SPATIAL_SCALE
    y1 = rois[:, 2:3] * _SPATIAL_SCALE
    x2 = rois[:, 3:4] * _SPATIAL_SCALE
    y2 = rois[:, 4:5] * _SPATIAL_SCALE
    roi_w = jnp.maximum(x2 - x1, 1.0)
    roi_h = jnp.maximum(y2 - y1, 1.0)
    bin_w = roi_w / P
    bin_h = roi_h / P

    S = _SAMPLING_RATIO
    jj = jax.lax.broadcasted_iota(jnp.int32, (G, P * S), 1)
    samp = (jj // S).astype(jnp.float32) + ((jj % S).astype(jnp.float32) + 0.5) / S
    ys = y1 + samp * bin_h
    xs = x1 + samp * bin_w

    Ay = _weights(ys, H, H)
    Bx = _weights(xs, W, W)

    noh = jax.lax.broadcasted_iota(jnp.float32, (G, N), 1)
    oh = (noh == b).astype(jnp.float32)
    Aoh = (Ay[:, :, None, :] * oh[:, None, :, None]).reshape(G * P, N * H)

    tmp = jax.lax.dot(Aoh, feat_ref[...], preferred_element_type=jnp.float32)
    tmp4 = tmp.reshape(G, P, W, C)

    out = jax.lax.dot_general(
        tmp4, Bx,
        dimension_numbers=(((2,), (2,)), ((0,), (0,))),
        preferred_element_type=jnp.float32,
    )
    out_ref[...] = out


def kernel(input, rois):
    N, C, H, W = input.shape
    K = rois.shape[0]
    P = _OUTPUT_SIZE
    G = _G
    Kp = ((K + G - 1) // G) * G

    feat_r = input.transpose(0, 2, 3, 1).reshape(N * H, W * C)
    rois_p = jnp.zeros((Kp, 5), jnp.float32).at[:K].set(rois)

    out = pl.pallas_call(
        _roi_kernel,
        grid=(Kp // G,),
        in_specs=[
            pl.BlockSpec((G, 5), lambda i: (i, 0)),
            pl.BlockSpec((N * H, W * C), lambda i: (0, 0)),
        ],
        out_specs=pl.BlockSpec((G, P, C, P), lambda i: (i, 0, 0, 0)),
        out_shape=jax.ShapeDtypeStruct((Kp, P, C, P), jnp.float32),
    )(rois_p, feat_r)

    return out[:K].transpose(0, 2, 1, 3)


# speedup vs baseline: 7.7564x; 7.7564x over previous
import jax
import jax.numpy as jnp
from jax.experimental import pallas as pl

_P = 7
_SCALE = 0.25
_S = 2
_G = 32


def _roi_kernel(rois_ref, feat_ref, out_ref):
    G, P, S = _G, _P, _S
    N, H, W, C = 4, 56, 56, 256
    R = G * P

    rois = rois_ref[...]
    b = rois[:, 0:1]
    x1 = rois[:, 1:2] * _SCALE
    y1 = rois[:, 2:3] * _SCALE
    x2 = rois[:, 3:4] * _SCALE
    y2 = rois[:, 4:5] * _SCALE
    bin_w = jnp.maximum(x2 - x1, 1.0) * (1.0 / P)
    bin_h = jnp.maximum(y2 - y1, 1.0) * (1.0 / P)

    def wmat(origin, binsz, cols, limit, pos_from_col):
        rowi = jax.lax.broadcasted_iota(jnp.int32, (R, cols), 0)
        pf = (rowi % P).astype(jnp.float32)
        colq = jax.lax.broadcasted_iota(jnp.int32, (R, cols), 1)
        pos = pos_from_col(colq)
        acc = jnp.zeros((R, cols), jnp.float32)
        for s in range(S):
            c = origin + (pf + (s + 0.5) / S) * binsz
            valid = (c >= -1.0) & (c <= float(limit))
            cc = jnp.clip(c, 0.0, float(limit - 1))
            w = jnp.maximum(1.0 - jnp.abs(pos - cc), 0.0)
            acc = acc + jnp.where(valid, w, 0.0)
        return acc * (1.0 / S)

    def ypos(colq):
        return (colq % H).astype(jnp.float32)

    A = wmat(y1, bin_h, N * H, H, ypos)
    colq = jax.lax.broadcasted_iota(jnp.int32, (R, N * H), 1)
    A = jnp.where((colq // H).astype(jnp.float32) == b, A, 0.0)

    tmp = jax.lax.dot(A, feat_ref[...], preferred_element_type=jnp.float32)

    B = wmat(x1, bin_w, W, W, lambda q: q.astype(jnp.float32))

    tmp4 = tmp.reshape(G, P, W, C)
    Bx = B.reshape(G, P, W)
    out = jax.lax.dot_general(
        Bx, tmp4,
        dimension_numbers=(((2,), (2,)), ((0,), (0,))),
        preferred_element_type=jnp.float32,
    )
    out_ref[...] = out


def kernel(input, rois):
    N, C, H, W = input.shape
    K = rois.shape[0]
    P, G = _P, _G
    Kp = ((K + G - 1) // G) * G

    feat_r = input.transpose(0, 2, 3, 1).reshape(N * H, W * C)
    rois_p = jnp.zeros((Kp, 5), jnp.float32).at[:K].set(rois)
    rois_rep = jnp.repeat(rois_p, P, axis=0)

    out = pl.pallas_call(
        _roi_kernel,
        grid=(Kp // G,),
        in_specs=[
            pl.BlockSpec((G * P, 5), lambda i: (i, 0)),
            pl.BlockSpec((N * H, W * C), lambda i: (0, 0)),
        ],
        out_specs=pl.BlockSpec((G, P, P, C), lambda i: (i, 0, 0, 0)),
        out_shape=jax.ShapeDtypeStruct((Kp, P, P, C), jnp.float32),
    )(rois_rep, feat_r)

    return out[:K].transpose(0, 3, 2, 1)


# speedup vs baseline: 7.7959x; 1.0051x over previous
import jax
import jax.numpy as jnp
from jax.experimental import pallas as pl

_P = 7
_SCALE = 0.25
_S = 2
_G = 32
_GT = 64


def _roi_kernel(rois_ref, feat_ref, out_ref):
    G, P, S = _G, _P, _S
    N, H, W, C = 4, 56, 56, 256
    R = G * P

    rois = rois_ref[...]
    b = rois[:, 0:1]
    x1 = rois[:, 1:2] * _SCALE
    y1 = rois[:, 2:3] * _SCALE
    x2 = rois[:, 3:4] * _SCALE
    y2 = rois[:, 4:5] * _SCALE
    bin_w = jnp.maximum(x2 - x1, 1.0) * (1.0 / P)
    bin_h = jnp.maximum(y2 - y1, 1.0) * (1.0 / P)

    def wmat(origin, binsz, cols, limit, pos_from_col):
        rowi = jax.lax.broadcasted_iota(jnp.int32, (R, cols), 0)
        pf = (rowi % P).astype(jnp.float32)
        colq = jax.lax.broadcasted_iota(jnp.int32, (R, cols), 1)
        pos = pos_from_col(colq)
        acc = jnp.zeros((R, cols), jnp.float32)
        for s in range(S):
            c = origin + (pf + (s + 0.5) / S) * binsz
            valid = (c >= -1.0) & (c <= float(limit))
            cc = jnp.clip(c, 0.0, float(limit - 1))
            w = jnp.maximum(1.0 - jnp.abs(pos - cc), 0.0)
            acc = acc + jnp.where(valid, w, 0.0)
        return acc * (1.0 / S)

    B = wmat(x1, bin_w, N * W, W, lambda q: (q % W).astype(jnp.float32))
    colq = jax.lax.broadcasted_iota(jnp.int32, (R, N * W), 1)
    B = jnp.where((colq // W).astype(jnp.float32) == b, B, 0.0)

    tmp = jax.lax.dot(B, feat_ref[...], preferred_element_type=jnp.float32)

    A = wmat(y1, bin_h, H, H, lambda q: q.astype(jnp.float32))

    tmp4 = tmp.reshape(G, P, H, C)
    Ay = A.reshape(G, P, H)
    out = jax.lax.dot_general(
        Ay, tmp4,
        dimension_numbers=(((2,), (2,)), ((0,), (0,))),
        preferred_element_type=jnp.float32,
    )
    out_ref[...] = out


def _tr_kernel(x_ref, o_ref):
    o_ref[...] = jnp.swapaxes(x_ref[...], 1, 2)


def kernel(input, rois):
    N, C, H, W = input.shape
    K = rois.shape[0]
    P, G = _P, _G

    feat_x = input.transpose(0, 3, 2, 1).reshape(N * W, H * C)
    rois_rep = jnp.repeat(rois, P, axis=0)

    out = pl.pallas_call(
        _roi_kernel,
        grid=((K + G - 1) // G,),
        in_specs=[
            pl.BlockSpec((G * P, 5), lambda i: (i, 0)),
            pl.BlockSpec((N * W, H * C), lambda i: (0, 0)),
        ],
        out_specs=pl.BlockSpec((G, P, P, C), lambda i: (i, 0, 0, 0)),
        out_shape=jax.ShapeDtypeStruct((K, P, P, C), jnp.float32),
    )(rois_rep, feat_x)

    out2 = pl.pallas_call(
        _tr_kernel,
        grid=((K + _GT - 1) // _GT,),
        in_specs=[pl.BlockSpec((_GT, P * P, C), lambda i: (i, 0, 0))],
        out_specs=pl.BlockSpec((_GT, C, P * P), lambda i: (i, 0, 0)),
        out_shape=jax.ShapeDtypeStruct((K, C, P * P), jnp.float32),
    )(out.reshape(K, P * P, C))

    return out2.reshape(K, C, P, P)
